# baseline (speedup 1.0000x reference)
import functools

import jax
import jax.numpy as jnp
from jax import lax
from jax.experimental import pallas as pl
from jax.experimental.pallas import tpu as pltpu
from jax.experimental.pallas import tpu_sc as plsc

EMBED = 32
IDX_COLS = 128
STEP_ROWS = 8
NBUF = 2


def _make_gather(n_rows):
    info = plsc.get_sparse_core_info()
    nc, ns = info.num_cores, info.num_subcores
    nw = nc * ns
    rows_per_w = n_rows // nw
    steps = rows_per_w // STEP_ROWS
    chunk = STEP_ROWS * IDX_COLS

    mesh = plsc.VectorSubcoreMesh(core_axis_name="c", subcore_axis_name="s")

    @functools.partial(
        pl.kernel,
        mesh=mesh,
        compiler_params=pltpu.CompilerParams(use_tc_tiling_on_sc=False),
        out_type=jax.ShapeDtypeStruct((n_rows * IDX_COLS, EMBED), jnp.float32),
        scratch_types=[
            pltpu.VMEM((NBUF, STEP_ROWS, IDX_COLS), jnp.int32),
            pltpu.VMEM((NBUF, chunk, EMBED), jnp.float32),
            pltpu.SemaphoreType.DMA((NBUF,)),
            pltpu.SemaphoreType.DMA((NBUF,)),
            pltpu.SemaphoreType.DMA((NBUF,)),
        ],
    )
    def k(table_hbm, idx_hbm, out_hbm, idx_v, rows_v, sem_idx, sem_g, sem_out):
        wid = lax.axis_index("s") * nc + lax.axis_index("c")
        row_base = wid * rows_per_w

        def start_idx(g, b):
            r0 = row_base + g * STEP_ROWS
            pltpu.async_copy(
                idx_hbm.at[pl.ds(r0, STEP_ROWS), :], idx_v.at[b], sem_idx.at[b]
            )

        def wait_idx(b):
            pltpu.make_async_copy(
                idx_hbm.at[pl.ds(0, STEP_ROWS), :], idx_v.at[b], sem_idx.at[b]
            ).wait()

        def start_out(g, b):
            r0 = row_base + g * STEP_ROWS
            pltpu.async_copy(
                rows_v.at[b], out_hbm.at[pl.ds(r0 * IDX_COLS, chunk), :],
                sem_out.at[b],
            )

        def wait_out(b):
            pltpu.make_async_copy(
                rows_v.at[b], out_hbm.at[pl.ds(0, chunk), :], sem_out.at[b]
            ).wait()

        def run_step(g, b, prefetch_g):
            if prefetch_g is not None:
                start_idx(prefetch_g, (b + 1) % NBUF)
            wait_idx(b)
            cps = [
                pltpu.async_copy(
                    table_hbm.at[idx_v.at[b, j]],
                    rows_v.at[b, pl.ds(j * IDX_COLS, IDX_COLS), :],
                    sem_g.at[b],
                )
                for j in range(STEP_ROWS)
            ]
            for cp in cps:
                cp.wait()
            start_out(g, b)

        start_idx(0, 0)

        @pl.loop(0, steps // NBUF)
        def _(i):
            g0 = i * NBUF
            pl.when(i > 0)(lambda: wait_out(0))
            run_step(g0, 0, g0 + 1)
            pl.when(i > 0)(lambda: wait_out(1))

            def step1_with_prefetch():
                run_step(g0 + 1, 1, g0 + 2)

            def step1_last():
                run_step(g0 + 1, 1, None)

            pl.when(i < steps // NBUF - 1)(step1_with_prefetch)
            pl.when(i == steps // NBUF - 1)(step1_last)

        wait_out(0)
        wait_out(1)

    return k


def kernel(location, table):
    b, h = location.shape
    idx = location.reshape(-1).astype(jnp.int32).reshape(-1, IDX_COLS)
    out = _make_gather(idx.shape[0])(table, idx)
    return out.reshape(b, h, EMBED)


# speedup vs baseline: 1.1223x; 1.1223x over previous
import functools

import jax
import jax.numpy as jnp
from jax import lax
from jax.experimental import pallas as pl
from jax.experimental.pallas import tpu as pltpu
from jax.experimental.pallas import tpu_sc as plsc

VOCAB_ROWS = 202
EMBED = 32
BLK_B = 128
NBUF = 2


def _make_lookup(n_b, n_h):
    info = plsc.get_sparse_core_info()
    nc, ns = info.num_cores, info.num_subcores
    nw = nc * ns
    b_per_w = n_b // nw
    blocks = b_per_w // BLK_B
    dt_n, di_n = EMBED // 8, 8

    mesh = plsc.VectorSubcoreMesh(core_axis_name="c", subcore_axis_name="s")

    @functools.partial(
        pl.kernel,
        mesh=mesh,
        compiler_params=pltpu.CompilerParams(
            use_tc_tiling_on_sc=False, needs_layout_passes=False
        ),
        out_type=jax.ShapeDtypeStruct((n_h, dt_n, n_b // BLK_B, di_n, BLK_B),
                                      jnp.float32),
        scratch_types=[
            pltpu.VMEM((VOCAB_ROWS, EMBED), jnp.float32),
            pltpu.VMEM((BLK_B, n_h), jnp.int32),
            pltpu.VMEM((NBUF, dt_n, di_n, BLK_B), jnp.float32),
            pltpu.SemaphoreType.DMA((NBUF,)),
        ],
    )
    def k(table_hbm, loc_hbm, out_hbm, table_v, loc_v, out_v, sem_out):
        wid = lax.axis_index("s") * nc + lax.axis_index("c")
        pltpu.sync_copy(table_hbm, table_v)
        iota16 = lax.iota(jnp.int32, 16)

        def wait_out(buf):
            pltpu.make_async_copy(
                out_v.at[buf], out_hbm.at[0, :, 0, :, :], sem_out.at[buf]
            ).wait()

        def process_h(h, bt, buf):
            idxs = [
                plsc.load_gather(
                    loc_v, [iota16 + (j * 16), jnp.full((16,), h, jnp.int32)]
                )
                for j in range(BLK_B // 16)
            ]
            for d in range(EMBED):
                dsplat = jnp.full((16,), d, jnp.int32)
                for j in range(BLK_B // 16):
                    v = plsc.load_gather(table_v, [idxs[j], dsplat])
                    out_v[buf, d // 8, d % 8, pl.ds(j * 16, 16)] = v
            pltpu.async_copy(
                out_v.at[buf], out_hbm.at[h, :, bt, :, :], sem_out.at[buf]
            )

        @pl.loop(0, blocks)
        def _(blk_i):
            bt = wid * blocks + blk_i
            pltpu.sync_copy(loc_hbm.at[pl.ds(bt * BLK_B, BLK_B), :], loc_v)

            @pl.loop(0, n_h // NBUF)
            def _(i):
                h0 = i * NBUF
                not_first = (blk_i + i) > 0
                pl.when(not_first)(lambda: wait_out(0))
                process_h(h0, bt, 0)
                pl.when(not_first)(lambda: wait_out(1))
                process_h(h0 + 1, bt, 1)

        wait_out(0)
        wait_out(1)

    return k


def kernel(location, table):
    b, h = location.shape
    q = _make_lookup(b, h)(table, location.astype(jnp.int32))
    return q.transpose(2, 4, 0, 1, 3).reshape(b, h, EMBED)


# speedup vs baseline: 2.0850x; 1.8578x over previous
import functools

import jax
import jax.numpy as jnp
from jax import lax
from jax.experimental import pallas as pl
from jax.experimental.pallas import tpu as pltpu
from jax.experimental.pallas import tpu_sc as plsc

VOCAB_ROWS = 202
EMBED = 32
BLK_B = 128
NBUF = 2
TILE = 8 * BLK_B


def _make_lookup(n_b, n_h):
    info = plsc.get_sparse_core_info()
    nc, ns = info.num_cores, info.num_subcores
    nw = nc * ns
    b_per_w = n_b // nw
    blocks = b_per_w // BLK_B
    dt_n = EMBED // 8
    n_bt = n_b // BLK_B

    mesh = plsc.VectorSubcoreMesh(core_axis_name="c", subcore_axis_name="s")

    @functools.partial(
        pl.kernel,
        mesh=mesh,
        compiler_params=pltpu.CompilerParams(
            use_tc_tiling_on_sc=False, needs_layout_passes=False
        ),
        out_type=jax.ShapeDtypeStruct((n_h, dt_n, n_bt * TILE), jnp.float32),
        scratch_types=[
            pltpu.VMEM((VOCAB_ROWS, EMBED), jnp.float32),
            pltpu.VMEM((BLK_B, n_h), jnp.int32),
            pltpu.VMEM((NBUF, dt_n * TILE), jnp.float32),
            pltpu.SemaphoreType.DMA((NBUF,)),
        ],
    )
    def k(table_hbm, loc_hbm, out_hbm, table_v, loc_v, out_v, sem_out):
        wid = lax.axis_index("s") * nc + lax.axis_index("c")
        pltpu.sync_copy(table_hbm, table_v)
        iota16 = lax.iota(jnp.int32, 16)

        def wait_out(buf):
            pltpu.make_async_copy(
                out_v.at[buf], out_hbm.at[0, 0, pl.ds(0, dt_n * TILE)],
                sem_out.at[buf],
            ).wait()

        def process_h(h, bt, buf):
            idxs = [
                plsc.load_gather(
                    loc_v, [iota16 + (j * 16), jnp.full((16,), h, jnp.int32)]
                )
                for j in range(BLK_B // 16)
            ]

            @plsc.parallel_loop(0, EMBED, unroll=4)
            def _(d):
                dsplat = jnp.full((16,), 1, jnp.int32) * d
                for j in range(BLK_B // 16):
                    v = plsc.load_gather(table_v, [idxs[j], dsplat])
                    out_v[buf, pl.ds(d * BLK_B + j * 16, 16)] = v

            for dt in range(dt_n):
                pltpu.async_copy(
                    out_v.at[buf, pl.ds(dt * TILE, TILE)],
                    out_hbm.at[h, dt, pl.ds(bt * TILE, TILE)],
                    sem_out.at[buf],
                )

        @pl.loop(0, blocks)
        def _(blk_i):
            bt = wid * blocks + blk_i
            pltpu.sync_copy(loc_hbm.at[pl.ds(bt * BLK_B, BLK_B), :], loc_v)

            @pl.loop(0, n_h // NBUF)
            def _(i):
                h0 = i * NBUF
                not_first = (blk_i + i) > 0
                pl.when(not_first)(lambda: wait_out(0))
                process_h(h0, bt, 0)
                pl.when(not_first)(lambda: wait_out(1))
                process_h(h0 + 1, bt, 1)

        wait_out(0)
        wait_out(1)

    return k


def kernel(location, table):
    b, h = location.shape
    q = _make_lookup(b, h)(table, location.astype(jnp.int32))
    q5 = q.reshape(h, EMBED // 8, b // BLK_B, 8, BLK_B)
    return q5.transpose(2, 4, 0, 1, 3).reshape(b, h, EMBED)
